# Initial kernel scaffold; baseline (speedup 1.0000x reference)
import jax
import jax.numpy as jnp
from jax.experimental import pallas as pl


def kernel(normu, keep, ignore):
    raise NotImplementedError("write your pallas kernel here")


# speedup vs baseline: 2.3776x; 2.3776x over previous
import functools

import jax
import jax.numpy as jnp
from jax import lax
from jax.experimental import pallas as pl
from jax.experimental.pallas import tpu as pltpu
from jax.experimental.pallas import tpu_sc as plsc

C = 256
HW = 65536
N = 16384
NC = 2
NS = 16
NW = NC * NS
COLS_PER_W = HW // NW
K_PER_W = N // NW
CHUNK = 128
N_CHUNKS = K_PER_W // CHUNK
JB = 512


def _gather16(a, i):
    dn = lax.GatherDimensionNumbers(
        offset_dims=(), collapsed_slice_dims=(0,), start_index_map=(0,))
    return lax.gather(a, i[:, None], dn, (1,),
                      mode=lax.GatherScatterMode.PROMISE_IN_BOUNDS)


def _tr_body(x_ref, o_ref):
    o_ref[...] = x_ref[...].T


def _transpose_keep(keep2):
    return pl.pallas_call(
        _tr_body,
        grid=(N // 256,),
        in_specs=[pl.BlockSpec((C, 256), lambda i: (0, i))],
        out_specs=pl.BlockSpec((256, C), lambda i: (i, 0)),
        out_shape=jax.ShapeDtypeStruct((N, C), jnp.float32),
    )(keep2)


def _winner_body(ign_hbm, w_hbm, flag_hbm, ign_v, w_v, flag_v):
    wid = lax.axis_index("s") * NC + lax.axis_index("c")
    lo = wid * COLS_PER_W
    pltpu.sync_copy(ign_hbm, ign_v)

    def zero_body(i, _):
        flag_v[pl.ds(i * 16, 16)] = jnp.zeros((16,), jnp.float32)
        w_v[pl.ds(i * 16, 16)] = jnp.zeros((16,), jnp.int32)
        return 0

    lax.fori_loop(0, COLS_PER_W // 16, zero_body, 0, unroll=False)

    iota = lax.iota(jnp.int32, 16)
    ones = jnp.ones((16,), jnp.float32)
    lane_masks = [iota == lane for lane in range(16)]

    def body(i, _):
        idx = ign_v[pl.ds(i * 16, 16)]
        k = i * 16 + iota
        inr = (idx >= lo) & (idx < lo + COLS_PER_W)
        loc = jnp.clip(idx - lo, 0, COLS_PER_W - 1)
        plsc.store_scatter(flag_v, [loc], ones, mask=inr)
        for lane in range(16):
            plsc.store_scatter(w_v, [loc], k, mask=inr & lane_masks[lane])
        return 0

    lax.fori_loop(0, N // 16, body, 0, unroll=False)
    pltpu.sync_copy(w_v, w_hbm.at[pl.ds(lo, COLS_PER_W)])
    pltpu.sync_copy(flag_v, flag_hbm.at[pl.ds(lo, COLS_PER_W)])


def _winner_call(ignore):
    mesh = plsc.VectorSubcoreMesh(core_axis_name="c", subcore_axis_name="s")
    f = pl.kernel(
        _winner_body,
        out_type=(
            jax.ShapeDtypeStruct((HW,), jnp.int32),
            jax.ShapeDtypeStruct((HW,), jnp.float32),
        ),
        mesh=mesh,
        scratch_types=[
            pltpu.VMEM((N,), jnp.int32),
            pltpu.VMEM((COLS_PER_W,), jnp.int32),
            pltpu.VMEM((COLS_PER_W,), jnp.float32),
        ],
        compiler_params=pltpu.CompilerParams(needs_layout_passes=False),
    )
    return f(ignore)


def _scatter_body(keep_t_hbm, ign_hbm, w_hbm, patch_hbm,
                  idx_v, wk_v, rows_v, sem):
    wid = lax.axis_index("s") * NC + lax.axis_index("c")
    pltpu.sync_copy(ign_hbm.at[wid], idx_v)
    for cnk in range(N_CHUNKS):
        idxc = idx_v.at[cnk]
        pltpu.async_copy(w_hbm.at[idxc], wk_v.at[cnk], sem).wait()
        pltpu.async_copy(keep_t_hbm.at[wk_v.at[cnk]], rows_v, sem).wait()
        pltpu.async_copy(rows_v, patch_hbm.at[idxc], sem).wait()


def _scatter_call(keep_t, ignore3, w):
    mesh = plsc.VectorSubcoreMesh(core_axis_name="c", subcore_axis_name="s")
    f = pl.kernel(
        _scatter_body,
        out_type=jax.ShapeDtypeStruct((HW, C), jnp.float32),
        mesh=mesh,
        scratch_types=[
            pltpu.VMEM((N_CHUNKS, CHUNK), jnp.int32),
            pltpu.VMEM((N_CHUNKS, CHUNK), jnp.int32),
            pltpu.VMEM((CHUNK, C), jnp.float32),
            pltpu.SemaphoreType.DMA,
        ],
        compiler_params=pltpu.CompilerParams(needs_layout_passes=False),
    )
    return f(keep_t, ignore3, w)


def _merge_body(n_ref, p_ref, f_ref, o_ref):
    f = f_ref[0]
    o_ref[...] = jnp.where(f != 0.0, p_ref[...].T, n_ref[...])


def _merge_call(normu2, patch, flag2):
    return pl.pallas_call(
        _merge_body,
        grid=(HW // JB,),
        in_specs=[
            pl.BlockSpec((C, JB), lambda j: (0, j)),
            pl.BlockSpec((JB, C), lambda j: (j, 0)),
            pl.BlockSpec((1, 1, JB), lambda j: (j, 0, 0)),
        ],
        out_specs=pl.BlockSpec((C, JB), lambda j: (0, j)),
        out_shape=jax.ShapeDtypeStruct((C, HW), jnp.float32),
    )(normu2, patch, flag2)


def kernel(normu, keep, ignore):
    normu2 = normu.reshape(C, HW)
    keep2 = keep.reshape(C, N)
    keep_t = _transpose_keep(keep2)
    w, flag = _winner_call(ignore)
    patch = _scatter_call(keep_t, ignore.reshape(NW, N_CHUNKS, CHUNK), w)
    out = _merge_call(normu2, patch, flag.reshape(HW // JB, 1, JB))
    return out.reshape(1, C, 256, 256)


# speedup vs baseline: 2.3835x; 1.0025x over previous
import functools

import jax
import jax.numpy as jnp
from jax import lax
from jax.experimental import pallas as pl
from jax.experimental.pallas import tpu as pltpu
from jax.experimental.pallas import tpu_sc as plsc

C = 256
HW = 65536
N = 16384
NC = 2
NS = 16
NW = NC * NS
COLS_PER_W = HW // NW
K_PER_W = N // NW
CHUNK = 128
N_CHUNKS = K_PER_W // CHUNK
JB = 512


def _gather16(a, i):
    dn = lax.GatherDimensionNumbers(
        offset_dims=(), collapsed_slice_dims=(0,), start_index_map=(0,))
    return lax.gather(a, i[:, None], dn, (1,),
                      mode=lax.GatherScatterMode.PROMISE_IN_BOUNDS)


def _tr_body(x_ref, o_ref):
    o_ref[...] = x_ref[...].T


def _transpose_keep(keep2):
    return pl.pallas_call(
        _tr_body,
        grid=(N // 256,),
        in_specs=[pl.BlockSpec((C, 256), lambda i: (0, i))],
        out_specs=pl.BlockSpec((256, C), lambda i: (i, 0)),
        out_shape=jax.ShapeDtypeStruct((N, C), jnp.float32),
    )(keep2)


def _winner_body(ign_hbm, w_hbm, flag_hbm, ign_v, w_v, flag_v):
    wid = lax.axis_index("s") * NC + lax.axis_index("c")
    lo = wid * COLS_PER_W
    pltpu.sync_copy(ign_hbm, ign_v)

    def zero_body(i, _):
        flag_v[pl.ds(i * 16, 16)] = jnp.zeros((16,), jnp.float32)
        w_v[pl.ds(i * 16, 16)] = jnp.zeros((16,), jnp.int32)
        return 0

    lax.fori_loop(0, COLS_PER_W // 16, zero_body, 0, unroll=False)

    iota = lax.iota(jnp.int32, 16)
    ones = jnp.ones((16,), jnp.float32)
    lane_masks = [iota == lane for lane in range(16)]

    def body(i, _):
        idx = ign_v[pl.ds(i * 16, 16)]
        k = i * 16 + iota
        inr = (idx >= lo) & (idx < lo + COLS_PER_W)
        loc = jnp.clip(idx - lo, 0, COLS_PER_W - 1)
        plsc.store_scatter(flag_v, [loc], ones, mask=inr)
        for lane in range(16):
            plsc.store_scatter(w_v, [loc], k, mask=inr & lane_masks[lane])
        return 0

    lax.fori_loop(0, N // 16, body, 0, unroll=False)
    pltpu.sync_copy(w_v, w_hbm.at[pl.ds(lo, COLS_PER_W)])
    pltpu.sync_copy(flag_v, flag_hbm.at[pl.ds(lo, COLS_PER_W)])


def _winner_call(ignore):
    mesh = plsc.VectorSubcoreMesh(core_axis_name="c", subcore_axis_name="s")
    f = pl.kernel(
        _winner_body,
        out_type=(
            jax.ShapeDtypeStruct((HW,), jnp.int32),
            jax.ShapeDtypeStruct((HW,), jnp.float32),
        ),
        mesh=mesh,
        scratch_types=[
            pltpu.VMEM((N,), jnp.int32),
            pltpu.VMEM((COLS_PER_W,), jnp.int32),
            pltpu.VMEM((COLS_PER_W,), jnp.float32),
        ],
        compiler_params=pltpu.CompilerParams(needs_layout_passes=False),
    )
    return f(ignore)


def _scatter_body(keep_t_hbm, ign_hbm, w_hbm, patch_hbm,
                  idx_v, wk_v, rows_v, sem):
    wid = lax.axis_index("s") * NC + lax.axis_index("c")
    pltpu.sync_copy(ign_hbm.at[wid], idx_v)
    for cnk in range(N_CHUNKS):
        idxc = idx_v.at[cnk]
        pltpu.async_copy(w_hbm.at[idxc], wk_v.at[cnk], sem).wait()
        pltpu.async_copy(keep_t_hbm.at[wk_v.at[cnk]], rows_v, sem).wait()
        pltpu.async_copy(rows_v, patch_hbm.at[idxc], sem).wait()


def _scatter_call(keep_t, ignore3, w):
    mesh = plsc.VectorSubcoreMesh(core_axis_name="c", subcore_axis_name="s")
    f = pl.kernel(
        _scatter_body,
        out_type=jax.ShapeDtypeStruct((HW, C), jnp.float32),
        mesh=mesh,
        scratch_types=[
            pltpu.VMEM((N_CHUNKS, CHUNK), jnp.int32),
            pltpu.VMEM((N_CHUNKS, CHUNK), jnp.int32),
            pltpu.VMEM((CHUNK, C), jnp.float32),
            pltpu.SemaphoreType.DMA,
        ],
        compiler_params=pltpu.CompilerParams(
            needs_layout_passes=False, use_tc_tiling_on_sc=True),
    )
    return f(keep_t, ignore3, w)


def _merge_body(n_ref, p_ref, f_ref, o_ref):
    f = f_ref[0]
    o_ref[...] = jnp.where(f != 0.0, p_ref[...].T, n_ref[...])


def _merge_call(normu2, patch, flag2):
    return pl.pallas_call(
        _merge_body,
        grid=(HW // JB,),
        in_specs=[
            pl.BlockSpec((C, JB), lambda j: (0, j)),
            pl.BlockSpec((JB, C), lambda j: (j, 0)),
            pl.BlockSpec((1, 1, JB), lambda j: (j, 0, 0)),
        ],
        out_specs=pl.BlockSpec((C, JB), lambda j: (0, j)),
        out_shape=jax.ShapeDtypeStruct((C, HW), jnp.float32),
    )(normu2, patch, flag2)


def kernel(normu, keep, ignore):
    normu2 = normu.reshape(C, HW)
    keep2 = keep.reshape(C, N)
    keep_t = _transpose_keep(keep2)
    w, flag = _winner_call(ignore)
    patch = _scatter_call(keep_t, ignore.reshape(NW, N_CHUNKS, CHUNK), w)
    out = _merge_call(normu2, patch, flag.reshape(HW // JB, 1, JB))
    return out.reshape(1, C, 256, 256)
